# baseline (speedup 1.0000x reference)
import functools

import jax
import jax.numpy as jnp
from jax import lax
from jax.experimental import pallas as pl
from jax.experimental.pallas import tpu as pltpu
from jax.experimental.pallas import tpu_sc as plsc

_NC = 2
_NS = 16
_NW = _NC * _NS


def kernel(inputs, table):
    del inputs
    rows, hidden = table.shape
    rpw = rows // _NW

    mesh = plsc.VectorSubcoreMesh(core_axis_name="c", subcore_axis_name="s")

    @functools.partial(
        pl.kernel,
        out_type=jax.ShapeDtypeStruct((rows, hidden), table.dtype),
        mesh=mesh,
    )
    def sc_copy(t_hbm, o_hbm):
        wid = lax.axis_index("s") * _NC + lax.axis_index("c")
        base = wid * rpw
        pltpu.sync_copy(t_hbm.at[pl.ds(base, rpw)], o_hbm.at[pl.ds(base, rpw)])

    return sc_copy(table)[None]


# speedup vs baseline: 24.8318x; 24.8318x over previous
import functools

import jax
import jax.numpy as jnp
from jax import lax
from jax.experimental import pallas as pl
from jax.experimental.pallas import tpu as pltpu
from jax.experimental.pallas import tpu_sc as plsc

_NC = 2
_NS = 16
_NW = _NC * _NS
_CH = 32


def kernel(inputs, table):
    del inputs
    rows, hidden = table.shape
    rpw = rows // _NW
    n_chunks = rpw // _CH

    mesh = plsc.VectorSubcoreMesh(core_axis_name="c", subcore_axis_name="s")

    @functools.partial(
        pl.kernel,
        out_type=jax.ShapeDtypeStruct((rows, hidden), table.dtype),
        mesh=mesh,
        scratch_types=[
            pltpu.VMEM((2, _CH, hidden), table.dtype),
            pltpu.SemaphoreType.DMA,
            pltpu.SemaphoreType.DMA,
        ],
    )
    def sc_copy(t_hbm, o_hbm, buf, sem_in, sem_out):
        wid = lax.axis_index("s") * _NC + lax.axis_index("c")
        base = wid * rpw
        in_flight = [None, None]
        out_flight = [None, None]
        in_flight[0] = pltpu.async_copy(
            t_hbm.at[pl.ds(base, _CH)], buf.at[0], sem_in)
        for i in range(n_chunks):
            b = i % 2
            nb = (i + 1) % 2
            if i + 1 < n_chunks:
                if out_flight[nb] is not None:
                    out_flight[nb].wait()
                in_flight[nb] = pltpu.async_copy(
                    t_hbm.at[pl.ds(base + (i + 1) * _CH, _CH)], buf.at[nb],
                    sem_in)
            in_flight[b].wait()
            out_flight[b] = pltpu.async_copy(
                buf.at[b], o_hbm.at[pl.ds(base + i * _CH, _CH)], sem_out)
        out_flight[(n_chunks - 1) % 2].wait()
        out_flight[n_chunks % 2].wait()

    return sc_copy(table)[None]


# speedup vs baseline: 24.9474x; 1.0047x over previous
import functools

import jax
import jax.numpy as jnp
from jax import lax
from jax.experimental import pallas as pl
from jax.experimental.pallas import tpu as pltpu
from jax.experimental.pallas import tpu_sc as plsc

_NC = 2
_NS = 16
_NW = _NC * _NS
_CH = 32
_NBUF = 3


def kernel(inputs, table):
    del inputs
    rows, hidden = table.shape
    rpw = rows // _NW
    n_chunks = rpw // _CH

    mesh = plsc.VectorSubcoreMesh(core_axis_name="c", subcore_axis_name="s")

    @functools.partial(
        pl.kernel,
        out_type=jax.ShapeDtypeStruct((rows, hidden), table.dtype),
        mesh=mesh,
        scratch_types=[
            pltpu.VMEM((_NBUF, _CH, hidden), table.dtype),
            pltpu.SemaphoreType.DMA((_NBUF,)),
            pltpu.SemaphoreType.DMA((_NBUF,)),
        ],
    )
    def sc_copy(t_hbm, o_hbm, buf, sem_in, sem_out):
        wid = lax.axis_index("s") * _NC + lax.axis_index("c")
        base = wid * rpw
        in_fl = [None] * _NBUF
        out_fl = [None] * _NBUF
        for j in range(min(2, n_chunks)):
            in_fl[j] = pltpu.async_copy(
                t_hbm.at[pl.ds(base + j * _CH, _CH)], buf.at[j],
                sem_in.at[j])
        for i in range(n_chunks):
            b = i % _NBUF
            pf = i + 2
            if pf < n_chunks:
                pb = pf % _NBUF
                if out_fl[pb] is not None:
                    out_fl[pb].wait()
                    out_fl[pb] = None
                in_fl[pb] = pltpu.async_copy(
                    t_hbm.at[pl.ds(base + pf * _CH, _CH)], buf.at[pb],
                    sem_in.at[pb])
            in_fl[b].wait()
            out_fl[b] = pltpu.async_copy(
                buf.at[b], o_hbm.at[pl.ds(base + i * _CH, _CH)],
                sem_out.at[b])
        for b in range(_NBUF):
            if out_fl[b] is not None:
                out_fl[b].wait()

    return sc_copy(table)[None]
